# baseline (speedup 1.0000x reference)
import functools

import jax
import jax.numpy as jnp
from jax import lax
from jax.experimental import pallas as pl
from jax.experimental.pallas import tpu as pltpu
from jax.experimental.pallas import tpu_sc as plsc


def _sc_gather(table, idx_list, ch):
    n = idx_list[0].shape[0]
    d = table.shape[1]
    k = len(idx_list)
    info = plsc.get_sparse_core_info()
    nw = info.num_cores * info.num_subcores
    b_per_w = n // nw
    n_chunks = b_per_w // ch
    nc = info.num_cores
    mesh = plsc.VectorSubcoreMesh(core_axis_name="c", subcore_axis_name="s")

    @functools.partial(
        pl.kernel,
        mesh=mesh,
        compiler_params=pltpu.CompilerParams(use_tc_tiling_on_sc=False),
        out_type=[jax.ShapeDtypeStruct((n, d), jnp.float32)] * k,
        scratch_types=(
            [pltpu.VMEM((2, ch), jnp.int32)] * k
            + [pltpu.VMEM((2, ch, d), jnp.float32)] * k
            + [pltpu.SemaphoreType.DMA] * 4
        ),
    )
    def gather_kernel(table_hbm, *rest):
        idx_hbms = rest[:k]
        outs = rest[k:2 * k]
        idx_vs = rest[2 * k:3 * k]
        buf_vs = rest[3 * k:4 * k]
        gsems = rest[4 * k:4 * k + 2]
        wsems = rest[4 * k + 2:4 * k + 4]
        wid = lax.axis_index("s") * nc + lax.axis_index("c")
        w_base = wid * b_per_w
        gathers = [None, None]
        writebacks = [None, None]

        for c in range(n_chunks + 1):
            s = c % 2
            if c >= 2 and writebacks[s] is not None:
                for cp in writebacks[s]:
                    cp.wait()
                writebacks[s] = None
            if c < n_chunks:
                base = w_base + c * ch
                for j in range(k):
                    pltpu.sync_copy(idx_hbms[j].at[pl.ds(base, ch)],
                                    idx_vs[j].at[s])
                gathers[s] = [
                    pltpu.async_copy(table_hbm.at[idx_vs[j].at[s]],
                                     buf_vs[j].at[s], gsems[s])
                    for j in range(k)
                ]
            if c >= 1:
                sp = (c - 1) % 2
                for cp in gathers[sp]:
                    cp.wait()
                gathers[sp] = None
                pbase = w_base + (c - 1) * ch
                writebacks[sp] = [
                    pltpu.async_copy(buf_vs[j].at[sp],
                                     outs[j].at[pl.ds(pbase, ch)], wsems[sp])
                    for j in range(k)
                ]
        for s in range(2):
            if writebacks[s] is not None:
                for cp in writebacks[s]:
                    cp.wait()

    res = gather_kernel(table, *idx_list)
    return list(res) if isinstance(res, (list, tuple)) else [res]


_PACK = 8


def _tc_dense(e_lhs_p, e_rhs_p, e_str_p, op_p, g_p, val_p,
              bd_lhs, bd_rhs, bd_str, bd_q, bd_t, bd_w2,
              r8, r32, w1_t, b1n_t, iota_t, cnum_t, b1_t, b2_t, out_d):
    s_tot = e_lhs_p.shape[0]
    s_blk = 4096
    grid = (s_tot // s_blk,)

    def body(lhs_ref, rhs_ref, str_ref, op_ref, g_ref, val_ref,
             bd_lhs_ref, bd_rhs_ref, bd_str_ref, bd_q_ref, bd_t_ref,
             bd_w2_ref, r8_ref, r32_ref, w1_ref, b1n_ref, iota_ref,
             cnum_ref, b1_ref, b2_ref, out_ref):
        dot = lambda a, b: jnp.dot(a, b, preferred_element_type=jnp.float32)
        val_rep = dot(val_ref[...], r8_ref[...])
        h = jnp.maximum(val_rep * w1_ref[...] + b1n_ref[...], 0.0)
        r_const = (dot(h, bd_q_ref[...])
                   + dot(str_ref[...], bd_str_ref[...])
                   + cnum_ref[...])
        r_col = dot(rhs_ref[...], bd_rhs_ref[...])
        op_rep = dot(op_ref[...].astype(jnp.float32), r32_ref[...])
        onehot = (jnp.abs(op_rep - iota_ref[...]) < 0.5).astype(jnp.float32)
        g_rep = dot(g_ref[...].astype(jnp.float32), r32_ref[...])
        pre = (dot(onehot, bd_t_ref[...])
               + dot(lhs_ref[...], bd_lhs_ref[...])
               + jnp.where(g_rep > 0.5, r_col, r_const)
               + b1_ref[...])
        out_ref[...] = dot(jnp.maximum(pre, 0.0), bd_w2_ref[...]) + b2_ref[...]

    row_spec = lambda d: pl.BlockSpec((s_blk, d), lambda i: (i, 0))
    full_spec = lambda a: pl.BlockSpec(a.shape, lambda i: (0,) * a.ndim)
    return pl.pallas_call(
        body,
        grid=grid,
        in_specs=[
            row_spec(e_lhs_p.shape[1]), row_spec(e_rhs_p.shape[1]),
            row_spec(e_str_p.shape[1]),
            row_spec(_PACK), row_spec(_PACK), row_spec(_PACK),
            full_spec(bd_lhs), full_spec(bd_rhs), full_spec(bd_str),
            full_spec(bd_q), full_spec(bd_t), full_spec(bd_w2),
            full_spec(r8), full_spec(r32), full_spec(w1_t), full_spec(b1n_t),
            full_spec(iota_t), full_spec(cnum_t), full_spec(b1_t),
            full_spec(b2_t),
        ],
        out_specs=row_spec(_PACK * out_d),
        out_shape=jax.ShapeDtypeStruct((s_tot, _PACK * out_d), jnp.float32),
    )(e_lhs_p, e_rhs_p, e_str_p, op_p, g_p, val_p,
      bd_lhs, bd_rhs, bd_str, bd_q, bd_t, bd_w2,
      r8, r32, w1_t, b1n_t, iota_t, cnum_t, b1_t, b2_t)


def kernel(op_type_id, lhs_col_id, rhs_is_col, rhs_col_id, rhs_lit_is_num,
           rhs_lit_val, rhs_lit_bucket, op_emb, col_emb, str_emb,
           W_num1, b_num1, W_num2, b_num2, W_const,
           W_fuse1, b_fuse1, W_fuse2, b_fuse2):
    n = op_type_id.shape[0]
    del rhs_lit_is_num
    p = _PACK
    op_d = op_emb.shape[1]
    col_d = col_emb.shape[1]
    str_d = str_emb.shape[1]
    out_d = W_fuse2.shape[0]
    num_ops = op_emb.shape[0]

    e_lhs, e_rhs = _sc_gather(col_emb, [lhs_col_id, rhs_col_id], ch=1280)
    (e_str,) = _sc_gather(str_emb, [rhs_lit_bucket], ch=2560)

    a_op = W_fuse1[:, :op_d]
    a_lhs = W_fuse1[:, op_d:op_d + col_d]
    a_rhs = W_fuse1[:, op_d + col_d:]
    m = a_rhs @ W_const
    m_num = m[:, :m.shape[1] - str_d]
    m_str = m[:, m.shape[1] - str_d:]
    q = m_num @ W_num2
    t_op = op_emb @ a_op.T
    eye = jnp.eye(p, dtype=jnp.float32)
    kron = jnp.kron
    bd_lhs = kron(eye, a_lhs.T)
    bd_rhs = kron(eye, a_rhs.T)
    bd_str = kron(eye, m_str.T)
    bd_q = kron(eye, q.T)
    bd_t = kron(eye, t_op)
    bd_w2 = kron(eye, W_fuse2.T)
    r8 = kron(eye, jnp.ones((1, W_num2.shape[0]), jnp.float32))
    r32 = kron(eye, jnp.ones((1, out_d), jnp.float32))
    tile = lambda v: jnp.tile(v.reshape(1, -1), (1, p))
    w1_t = tile(W_num1[:, 0])
    b1n_t = tile(b_num1)
    iota_t = jnp.tile(jnp.arange(num_ops, dtype=jnp.float32).reshape(1, -1),
                      (1, p))
    cnum_t = tile(b_num2 @ m_num.T)
    b1_t = tile(b_fuse1)
    b2_t = tile(b_fuse2)

    out_p = _tc_dense(
        e_lhs.reshape(n // p, p * col_d), e_rhs.reshape(n // p, p * col_d),
        e_str.reshape(n // p, p * str_d),
        op_type_id.reshape(n // p, p), rhs_is_col.reshape(n // p, p),
        rhs_lit_val.reshape(n // p, p),
        bd_lhs, bd_rhs, bd_str, bd_q, bd_t, bd_w2,
        r8, r32, w1_t, b1n_t, iota_t, cnum_t, b1_t, b2_t, out_d)
    out_t = jnp.transpose(out_p.reshape(n // p, p, out_d),
                          (2, 0, 1)).reshape(out_d, n)
    return out_t.T


# speedup vs baseline: 1.0053x; 1.0053x over previous
import functools

import jax
import jax.numpy as jnp
from jax import lax
from jax.experimental import pallas as pl
from jax.experimental.pallas import tpu as pltpu
from jax.experimental.pallas import tpu_sc as plsc


def _sc_gather(col_emb, str_emb, lhs_col_id, rhs_col_id, rhs_lit_bucket):
    n = lhs_col_id.shape[0]
    col_d = col_emb.shape[1]
    str_d = str_emb.shape[1]
    n_col = col_emb.shape[0]
    n_str = str_emb.shape[0]
    info = plsc.get_sparse_core_info()
    nw = info.num_cores * info.num_subcores
    b_per_w = n // nw
    ch = 1280
    n_chunks = b_per_w // ch
    nc = info.num_cores
    pk = 128 // col_d
    mesh = plsc.VectorSubcoreMesh(core_axis_name="c", subcore_axis_name="s")

    @functools.partial(
        pl.kernel,
        mesh=mesh,
        compiler_params=pltpu.CompilerParams(use_tc_tiling_on_sc=False),
        out_type=[
            jax.ShapeDtypeStruct((n, col_d), jnp.float32),
            jax.ShapeDtypeStruct((n, col_d), jnp.float32),
            jax.ShapeDtypeStruct((n, str_d), jnp.float32),
        ],
        scratch_types=[
            pltpu.VMEM((2, ch), jnp.int32),
            pltpu.VMEM((2, ch), jnp.int32),
            pltpu.VMEM((2, ch), jnp.int32),
            pltpu.VMEM((2, ch, col_d), jnp.float32),
            pltpu.VMEM((2, ch, col_d), jnp.float32),
            pltpu.VMEM((2, ch, str_d), jnp.float32),
            pltpu.SemaphoreType.DMA,
            pltpu.SemaphoreType.DMA,
            pltpu.SemaphoreType.DMA,
            pltpu.SemaphoreType.DMA,
        ],
    )
    def gather_kernel(col_hbm, str_hbm, ilhs_hbm, irhs_hbm, istr_hbm,
                      out_lhs, out_rhs, out_str,
                      il_v, ir_v, is_v, bl_v, br_v, bs_v,
                      gsem0, gsem1, wsem0, wsem1):
        wid = lax.axis_index("s") * nc + lax.axis_index("c")
        w_base = wid * b_per_w
        col_t = col_hbm
        str_t = str_hbm
        gsems = (gsem0, gsem1)
        wsems = (wsem0, wsem1)
        gathers = [None, None]
        writebacks = [None, None]

        for c in range(n_chunks + 1):
            s = c % 2
            if c >= 2 and writebacks[s] is not None:
                for cp in writebacks[s]:
                    cp.wait()
                writebacks[s] = None
            if c < n_chunks:
                base = w_base + c * ch
                pltpu.sync_copy(ilhs_hbm.at[pl.ds(base, ch)], il_v.at[s])
                pltpu.sync_copy(irhs_hbm.at[pl.ds(base, ch)], ir_v.at[s])
                pltpu.sync_copy(istr_hbm.at[pl.ds(base, ch)], is_v.at[s])
                gathers[s] = [
                    pltpu.async_copy(col_t.at[il_v.at[s]], bl_v.at[s],
                                     gsems[s]),
                    pltpu.async_copy(col_t.at[ir_v.at[s]], br_v.at[s],
                                     gsems[s]),
                    pltpu.async_copy(str_t.at[is_v.at[s]], bs_v.at[s],
                                     gsems[s]),
                ]
            if c >= 1:
                sp = (c - 1) % 2
                for cp in gathers[sp]:
                    cp.wait()
                gathers[sp] = None
                pbase = w_base + (c - 1) * ch
                writebacks[sp] = [
                    pltpu.async_copy(
                        bl_v.at[sp], out_lhs.at[pl.ds(pbase, ch)], wsems[sp]),
                    pltpu.async_copy(
                        br_v.at[sp], out_rhs.at[pl.ds(pbase, ch)], wsems[sp]),
                    pltpu.async_copy(
                        bs_v.at[sp], out_str.at[pl.ds(pbase, ch)], wsems[sp]),
                ]
        for s in range(2):
            if writebacks[s] is not None:
                for cp in writebacks[s]:
                    cp.wait()

    return gather_kernel(col_emb, str_emb,
                         lhs_col_id, rhs_col_id, rhs_lit_bucket)


_PACK = 8


def _tc_dense(e_lhs_p, e_rhs_p, e_str_p, op_p, g_p, val_p,
              bd_lhs, bd_rhs, bd_str, bd_q, bd_t, bd_w2,
              r8, r32, w1_t, b1n_t, iota_t, cnum_t, b1_t, b2_t, out_d):
    s_tot = e_lhs_p.shape[0]
    s_blk = 4096
    grid = (s_tot // s_blk,)

    def body(lhs_ref, rhs_ref, str_ref, op_ref, g_ref, val_ref,
             bd_lhs_ref, bd_rhs_ref, bd_str_ref, bd_q_ref, bd_t_ref,
             bd_w2_ref, r8_ref, r32_ref, w1_ref, b1n_ref, iota_ref,
             cnum_ref, b1_ref, b2_ref, out_ref):
        dot = lambda a, b: jnp.dot(a, b, preferred_element_type=jnp.float32)
        val_rep = dot(val_ref[...], r8_ref[...])
        h = jnp.maximum(val_rep * w1_ref[...] + b1n_ref[...], 0.0)
        r_const = (dot(h, bd_q_ref[...])
                   + dot(str_ref[...], bd_str_ref[...])
                   + cnum_ref[...])
        r_col = dot(rhs_ref[...], bd_rhs_ref[...])
        op_rep = dot(op_ref[...].astype(jnp.float32), r32_ref[...])
        onehot = (jnp.abs(op_rep - iota_ref[...]) < 0.5).astype(jnp.float32)
        g_rep = dot(g_ref[...].astype(jnp.float32), r32_ref[...])
        pre = (dot(onehot, bd_t_ref[...])
               + dot(lhs_ref[...], bd_lhs_ref[...])
               + jnp.where(g_rep > 0.5, r_col, r_const)
               + b1_ref[...])
        out_ref[...] = dot(jnp.maximum(pre, 0.0), bd_w2_ref[...]) + b2_ref[...]

    row_spec = lambda d: pl.BlockSpec((s_blk, d), lambda i: (i, 0))
    full_spec = lambda a: pl.BlockSpec(a.shape, lambda i: (0,) * a.ndim)
    return pl.pallas_call(
        body,
        grid=grid,
        in_specs=[
            row_spec(e_lhs_p.shape[1]), row_spec(e_rhs_p.shape[1]),
            row_spec(e_str_p.shape[1]),
            row_spec(_PACK), row_spec(_PACK), row_spec(_PACK),
            full_spec(bd_lhs), full_spec(bd_rhs), full_spec(bd_str),
            full_spec(bd_q), full_spec(bd_t), full_spec(bd_w2),
            full_spec(r8), full_spec(r32), full_spec(w1_t), full_spec(b1n_t),
            full_spec(iota_t), full_spec(cnum_t), full_spec(b1_t),
            full_spec(b2_t),
        ],
        out_specs=row_spec(_PACK * out_d),
        out_shape=jax.ShapeDtypeStruct((s_tot, _PACK * out_d), jnp.float32),
    )(e_lhs_p, e_rhs_p, e_str_p, op_p, g_p, val_p,
      bd_lhs, bd_rhs, bd_str, bd_q, bd_t, bd_w2,
      r8, r32, w1_t, b1n_t, iota_t, cnum_t, b1_t, b2_t)


def kernel(op_type_id, lhs_col_id, rhs_is_col, rhs_col_id, rhs_lit_is_num,
           rhs_lit_val, rhs_lit_bucket, op_emb, col_emb, str_emb,
           W_num1, b_num1, W_num2, b_num2, W_const,
           W_fuse1, b_fuse1, W_fuse2, b_fuse2):
    n = op_type_id.shape[0]
    del rhs_lit_is_num
    p = _PACK
    op_d = op_emb.shape[1]
    col_d = col_emb.shape[1]
    str_d = str_emb.shape[1]
    out_d = W_fuse2.shape[0]
    num_ops = op_emb.shape[0]

    e_lhs, e_rhs, e_str = _sc_gather(col_emb, str_emb, lhs_col_id,
                                     rhs_col_id, rhs_lit_bucket)

    a_op = W_fuse1[:, :op_d]
    a_lhs = W_fuse1[:, op_d:op_d + col_d]
    a_rhs = W_fuse1[:, op_d + col_d:]
    m = a_rhs @ W_const
    m_num = m[:, :m.shape[1] - str_d]
    m_str = m[:, m.shape[1] - str_d:]
    q = m_num @ W_num2
    t_op = op_emb @ a_op.T
    eye = jnp.eye(p, dtype=jnp.float32)
    kron = jnp.kron
    bd_lhs = kron(eye, a_lhs.T)
    bd_rhs = kron(eye, a_rhs.T)
    bd_str = kron(eye, m_str.T)
    bd_q = kron(eye, q.T)
    bd_t = kron(eye, t_op)
    bd_w2 = kron(eye, W_fuse2.T)
    r8 = kron(eye, jnp.ones((1, W_num2.shape[0]), jnp.float32))
    r32 = kron(eye, jnp.ones((1, out_d), jnp.float32))
    tile = lambda v: jnp.tile(v.reshape(1, -1), (1, p))
    w1_t = tile(W_num1[:, 0])
    b1n_t = tile(b_num1)
    iota_t = jnp.tile(jnp.arange(num_ops, dtype=jnp.float32).reshape(1, -1),
                      (1, p))
    cnum_t = tile(b_num2 @ m_num.T)
    b1_t = tile(b_fuse1)
    b2_t = tile(b_fuse2)

    out_p = _tc_dense(
        e_lhs.reshape(n // p, p * col_d), e_rhs.reshape(n // p, p * col_d),
        e_str.reshape(n // p, p * str_d),
        op_type_id.reshape(n // p, p), rhs_is_col.reshape(n // p, p),
        rhs_lit_val.reshape(n // p, p),
        bd_lhs, bd_rhs, bd_str, bd_q, bd_t, bd_w2,
        r8, r32, w1_t, b1n_t, iota_t, cnum_t, b1_t, b2_t, out_d)
    out_t = jnp.transpose(out_p.reshape(n // p, p, out_d),
                          (2, 0, 1)).reshape(out_d, n)
    return out_t.T
